# baseline (speedup 1.0000x reference)
import functools

import jax
import jax.numpy as jnp
from jax import lax
from jax.experimental import pallas as pl
from jax.experimental.pallas import tpu as pltpu
from jax.experimental.pallas import tpu_sc as plsc

NUM_ITERS = 4
THRESH = 1e-08

NSC = 2
NSUB = 16
NW = NSC * NSUB
CHUNK = 125
KG = 20
KM = 5
GROUPG = KG * CHUNK
GROUPM = KM * CHUNK


def _mesh():
    return plsc.VectorSubcoreMesh(
        core_axis_name="c", subcore_axis_name="s",
        num_cores=NSC, num_subcores=NSUB)


def _sc_params():
    return pltpu.CompilerParams(use_tc_tiling_on_sc=False)


def _sc_gather(table, idx2d, num_edges, num_classes):
    nrows = num_edges // CHUNK
    per_w = nrows // NW
    ng = per_w // KG
    npairs = ng // 2

    @functools.partial(
        pl.kernel,
        mesh=_mesh(),
        out_type=jax.ShapeDtypeStruct((num_edges, num_classes), jnp.float32),
        scratch_types=[
            pltpu.VMEM((per_w, CHUNK), jnp.int32),
            pltpu.VMEM((2, GROUPG, num_classes), jnp.float32),
            pltpu.SemaphoreType.DMA,
            pltpu.SemaphoreType.DMA,
            pltpu.SemaphoreType.DMA,
        ],
        compiler_params=_sc_params(),
    )
    def k(table_hbm, idx_hbm, out_hbm, idx_v, rows_v, gsem, osem0, osem1):
        wid = lax.axis_index("s") * NSC + lax.axis_index("c")
        row0 = wid * per_w
        ebase = row0 * CHUNK

        pltpu.sync_copy(idx_hbm.at[pl.ds(row0, per_w)], idx_v)

        def out_slice(g):
            return out_hbm.at[pl.ds(ebase + g * GROUPG, GROUPG)]

        def do_group(g, b, osem, first):
            if not first:
                pltpu.make_async_copy(rows_v.at[b], out_slice(g - 2),
                                      osem).wait()

            def fire(kk, c):
                pltpu.async_copy(
                    table_hbm.at[idx_v.at[g * KG + kk]],
                    rows_v.at[b, pl.ds(kk * CHUNK, CHUNK)], gsem)
                return c
            lax.fori_loop(0, KG, fire, 0)

            def drain(kk, c):
                pltpu.make_async_copy(
                    table_hbm.at[pl.ds(0, CHUNK)],
                    rows_v.at[b, pl.ds(kk * CHUNK, CHUNK)], gsem).wait()
                return c
            lax.fori_loop(0, KG, drain, 0)

            pltpu.async_copy(rows_v.at[b], out_slice(g), osem)

        do_group(0, 0, osem0, first=True)
        do_group(1, 1, osem1, first=True)

        def pair(p, c):
            do_group(2 * p, 0, osem0, first=False)
            do_group(2 * p + 1, 1, osem1, first=False)
            return c
        lax.fori_loop(1, npairs, pair, 0)

        pltpu.make_async_copy(rows_v.at[0], out_slice(ng - 2), osem0).wait()
        pltpu.make_async_copy(rows_v.at[1], out_slice(ng - 1), osem1).wait()

    return k(table, idx2d)


def _sc_mega(ln, rev2, src2, num_nodes, num_classes):
    num_edges = ln.shape[0]
    nrows = num_edges // CHUNK
    per_w = nrows // NW
    ng = per_w // KM
    npairs = ng // 2
    zrows = 125
    per_sub = num_nodes // NSUB

    @functools.partial(
        pl.kernel,
        mesh=_mesh(),
        out_type=(
            jax.ShapeDtypeStruct((num_edges, num_classes), jnp.float32),
            jax.ShapeDtypeStruct((num_nodes, num_classes), jnp.float32),
            jax.ShapeDtypeStruct((num_nodes, num_classes), jnp.float32),
            jax.ShapeDtypeStruct((NSC, num_edges, num_classes), jnp.float32),
        ),
        scratch_types=[
            pltpu.VMEM((per_w, CHUNK), jnp.int32),
            pltpu.VMEM((per_w, CHUNK), jnp.int32),
            pltpu.VMEM((2, GROUPM, num_classes), jnp.float32),
            pltpu.VMEM((zrows, num_classes), jnp.float32),
            pltpu.VMEM_SHARED((num_nodes, num_classes), jnp.float32),
            pltpu.SemaphoreType.DMA,
            pltpu.SemaphoreType.DMA,
            pltpu.SemaphoreType.DMA,
        ],
        compiler_params=_sc_params(),
    )
    def k(ln_hbm, rev_hbm, src_hbm, lnrev_hbm, p0_hbm, p1_hbm, g_hbm,
          irev_v, isrc_v, rows_v, zbuf_v, acc_sh,
          gsem, osem0, osem1):
        cid = lax.axis_index("c")
        sid = lax.axis_index("s")
        wid = sid * NSC + cid
        row0 = wid * per_w
        ebase = row0 * CHUNK

        def zfill(i, c):
            zbuf_v[i, :] = jnp.zeros((num_classes,), jnp.float32)
            return c
        lax.fori_loop(0, zrows, zfill, 0)

        def zcopy(kk, c):
            pltpu.sync_copy(
                zbuf_v, acc_sh.at[pl.ds(sid * per_sub + kk * zrows, zrows)])
            return c
        lax.fori_loop(0, per_sub // zrows, zcopy, 0)

        pltpu.sync_copy(rev_hbm.at[pl.ds(row0, per_w)], irev_v)
        pltpu.sync_copy(src_hbm.at[pl.ds(row0, per_w)], isrc_v)
        plsc.subcore_barrier()

        def out_slice(g):
            return lnrev_hbm.at[pl.ds(ebase + g * GROUPM, GROUPM)]

        def do_group(g, b, osem, first):
            if not first:
                pltpu.make_async_copy(rows_v.at[b], out_slice(g - 2),
                                      osem).wait()

            def fire(kk, c):
                pltpu.async_copy(
                    ln_hbm.at[irev_v.at[g * KM + kk]],
                    rows_v.at[b, pl.ds(kk * CHUNK, CHUNK)], gsem)
                return c
            lax.fori_loop(0, KM, fire, 0)

            def drain(kk, c):
                pltpu.make_async_copy(
                    ln_hbm.at[pl.ds(0, CHUNK)],
                    rows_v.at[b, pl.ds(kk * CHUNK, CHUNK)], gsem).wait()
                return c
            lax.fori_loop(0, KM, drain, 0)

            pltpu.async_copy(rows_v.at[b], out_slice(g), osem)

            def sfire(kk, c):
                pltpu.sync_copy(
                    rows_v.at[b, pl.ds(kk * CHUNK, CHUNK)],
                    acc_sh.at[isrc_v.at[g * KM + kk]], add=True)
                return c
            lax.fori_loop(0, KM, sfire, 0)

        do_group(0, 0, osem0, first=True)
        do_group(1, 1, osem1, first=True)

        def pair(p, c):
            do_group(2 * p, 0, osem0, first=False)
            do_group(2 * p + 1, 1, osem1, first=False)
            return c
        lax.fori_loop(1, npairs, pair, 0)

        pltpu.make_async_copy(rows_v.at[0], out_slice(ng - 2), osem0).wait()
        pltpu.make_async_copy(rows_v.at[1], out_slice(ng - 1), osem1).wait()

        plsc.subcore_barrier()

        @pl.when(cid == 0)
        def _():
            pltpu.sync_copy(
                acc_sh.at[pl.ds(sid * per_sub, per_sub)],
                p0_hbm.at[pl.ds(sid * per_sub, per_sub)])

        @pl.when(cid == 1)
        def _():
            pltpu.sync_copy(
                acc_sh.at[pl.ds(sid * per_sub, per_sub)],
                p1_hbm.at[pl.ds(sid * per_sub, per_sub)])
        plsc.subcore_barrier()

        pltpu.sync_copy(src_hbm.at[pl.ds((wid ^ 1) * per_w, per_w)], irev_v)

        def fgroup(part_hbm, idx_buf, fbase, g, b, osem, first):
            def gout_slice(gg):
                return g_hbm.at[cid, pl.ds(fbase + gg * GROUPM, GROUPM)]

            if not first:
                pltpu.make_async_copy(rows_v.at[b], gout_slice(g - 2),
                                      osem).wait()

            def fire(kk, c):
                pltpu.async_copy(
                    part_hbm.at[idx_buf.at[g * KM + kk]],
                    rows_v.at[b, pl.ds(kk * CHUNK, CHUNK)], gsem)
                return c
            lax.fori_loop(0, KM, fire, 0)

            def drain(kk, c):
                pltpu.make_async_copy(
                    part_hbm.at[pl.ds(0, CHUNK)],
                    rows_v.at[b, pl.ds(kk * CHUNK, CHUNK)], gsem).wait()
                return c
            lax.fori_loop(0, KM, drain, 0)

            pltpu.async_copy(rows_v.at[b], gout_slice(g), osem)

        def fphase(part_hbm, idx_buf, fbase):
            fgroup(part_hbm, idx_buf, fbase, 0, 0, osem0, first=True)
            fgroup(part_hbm, idx_buf, fbase, 1, 1, osem1, first=True)

            def fpair(p, c):
                fgroup(part_hbm, idx_buf, fbase, 2 * p, 0, osem0,
                       first=False)
                fgroup(part_hbm, idx_buf, fbase, 2 * p + 1, 1, osem1,
                       first=False)
                return c
            lax.fori_loop(1, npairs, fpair, 0)
            pltpu.make_async_copy(
                rows_v.at[0],
                g_hbm.at[cid, pl.ds(fbase + (ng - 2) * GROUPM, GROUPM)],
                osem0).wait()
            pltpu.make_async_copy(
                rows_v.at[1],
                g_hbm.at[cid, pl.ds(fbase + (ng - 1) * GROUPM, GROUPM)],
                osem1).wait()

        pbase = (wid ^ 1) * per_w * CHUNK

        @pl.when(cid == 0)
        def _():
            fphase(p0_hbm, isrc_v, ebase)
            fphase(p0_hbm, irev_v, pbase)

        @pl.when(cid == 1)
        def _():
            fphase(p1_hbm, isrc_v, ebase)
            fphase(p1_hbm, irev_v, pbase)

    return k(ln, rev2, src2)


def _tc_edge_math(gp, g, lnrev, potential, first):
    num_edges, num_classes = gp.shape
    be = 4000
    grid = num_edges // be

    def body(*refs):
        if first:
            gp_ref, pot_ref, out_ref = refs
            r = gp_ref[...]
        else:
            gp_ref, g_ref, lnrev_ref, pot_ref, out_ref = refs
            logits = jnp.log(gp_ref[...]) + g_ref[0] + g_ref[1]
            m = jnp.max(logits, axis=1, keepdims=True)
            e = jnp.exp(logits - m)
            bsrc = e / jnp.sum(e, axis=1, keepdims=True)
            r = bsrc * jnp.exp(-lnrev_ref[...])
        t = jnp.dot(r, pot_ref[...], preferred_element_type=jnp.float32)
        n = t / jnp.sum(t, axis=1, keepdims=True)
        out_ref[...] = jnp.log(n)

    espec = pl.BlockSpec((be, num_classes), lambda i: (i, 0))
    gspec = pl.BlockSpec((NSC, be, num_classes), lambda i: (0, i, 0))
    pspec = pl.BlockSpec((num_classes, num_classes), lambda i: (0, 0))
    if first:
        in_specs = [espec, pspec]
        args = (gp, potential)
    else:
        in_specs = [espec, gspec, espec, pspec]
        args = (gp, g, lnrev, potential)
    return pl.pallas_call(
        body,
        grid=(grid,),
        in_specs=in_specs,
        out_specs=espec,
        out_shape=jax.ShapeDtypeStruct((num_edges, num_classes), jnp.float32),
    )(*args)


def _tc_softmax(p0, p1, priors, old_beliefs):
    num_nodes, num_classes = priors.shape
    bn = 5000
    grid = num_nodes // bn

    def body(p0_ref, p1_ref, pri_ref, old_ref, bel_ref, dmax_ref):
        logits = jnp.log(pri_ref[...]) + p0_ref[...] + p1_ref[...]
        m = jnp.max(logits, axis=1, keepdims=True)
        e = jnp.exp(logits - m)
        b = e / jnp.sum(e, axis=1, keepdims=True)
        bel_ref[...] = b
        d = jnp.max(jnp.abs(b - old_ref[...]))
        dmax_ref[...] = jnp.full((8, 128), d, jnp.float32)

    nspec = pl.BlockSpec((bn, num_classes), lambda i: (i, 0))
    return pl.pallas_call(
        body,
        grid=(grid,),
        in_specs=[nspec, nspec, nspec, nspec],
        out_specs=[nspec, pl.BlockSpec((8, 128), lambda i: (i, 0))],
        out_shape=[
            jax.ShapeDtypeStruct((num_nodes, num_classes), jnp.float32),
            jax.ShapeDtypeStruct((grid * 8, 128), jnp.float32),
        ],
    )(p0, p1, priors, old_beliefs)


def kernel(priors, potential, src_nodes, dst_nodes, rev_edges):
    num_edges = src_nodes.shape[0]
    num_nodes, num_classes = priors.shape
    nrows = num_edges // CHUNK

    src2 = src_nodes.reshape(nrows, CHUNK)
    rev2 = rev_edges.reshape(nrows, CHUNK)
    del dst_nodes

    gp = _sc_gather(priors, src2, num_edges, num_classes)

    beliefs = priors
    lnrev_state = None
    g_state = None
    done = jnp.array(False)
    for it in range(NUM_ITERS):
        ln_new = _tc_edge_math(gp, g_state, lnrev_state, potential,
                               first=(it == 0))
        lnrev_new, p0, p1, g_new = _sc_mega(
            ln_new, rev2, src2, num_nodes, num_classes)
        b_new, bmax = _tc_softmax(p0, p1, priors, beliefs)
        diff = jnp.max(bmax)
        if it == 0:
            lnrev_state, g_state = lnrev_new, g_new
        else:
            lnrev_state = jnp.where(done, lnrev_state, lnrev_new)
            g_state = jnp.where(done, g_state, g_new)
        beliefs = jnp.where(done, beliefs, b_new)
        done = jnp.logical_or(done, diff < THRESH)
    return beliefs
